# baseline (speedup 1.0000x reference)
import math

import jax
import jax.numpy as jnp
from jax import lax
from jax.experimental import pallas as pl
from jax.experimental.pallas import tpu as pltpu
from jax.experimental.pallas import tpu_sc as plsc

D_MODEL = 128
SCALE = math.sqrt(D_MODEL)

NUM_CORES = 2
NUM_SUBCORES = 16
NW = NUM_CORES * NUM_SUBCORES

B_TOTAL = 4096 * 50
B_PER_W = B_TOTAL // NW
CHUNK = 64
N_CHUNKS = B_PER_W // CHUNK
NBUF = 2


def _emb_kernel(x_hbm, table_hbm, out_hbm,
                idx_v, gb0, gb1, ob0, ob1, gs0, gs1, os0, os1):
    gb = (gb0, gb1)
    ob = (ob0, ob1)
    gs = (gs0, gs1)
    osm = (os0, os1)

    wid = lax.axis_index("s") * NUM_CORES + lax.axis_index("c")

    pltpu.sync_copy(x_hbm.at[wid], idx_v)

    def g_desc(j, b):
        return pltpu.make_async_copy(table_hbm.at[idx_v.at[j]], gb[b], gs[b])

    def o_desc(j, b):
        return pltpu.make_async_copy(ob[b], out_hbm.at[wid * N_CHUNKS + j], osm[b])

    g_desc(0, 0).start()
    g_desc(1, 1).start()

    def step(j, b):
        g_desc(j, b).wait()

        @pl.when(j >= NBUF)
        def _():
            o_desc(j - NBUF, b).wait()

        @plsc.parallel_loop(0, CHUNK, unroll=4)
        def _(r):
            for c in range(D_MODEL // 16):
                ob[b][r, pl.ds(c * 16, 16)] = gb[b][r, pl.ds(c * 16, 16)] * SCALE

        @pl.when(j + NBUF < N_CHUNKS)
        def _():
            g_desc(j + NBUF, b).start()

        o_desc(j, b).start()

    @pl.loop(0, N_CHUNKS, step=NBUF)
    def _(j):
        for b in range(NBUF):
            step(j + b, b)

    o_desc(N_CHUNKS - NBUF, 0).wait()
    o_desc(N_CHUNKS - 1, 1).wait()


def kernel(x, emb_table):
    b_dim, s_dim = x.shape
    x_g = x.astype(jnp.int32).T.reshape(NW, N_CHUNKS, CHUNK)

    mesh = plsc.VectorSubcoreMesh(core_axis_name="c", subcore_axis_name="s")
    out = pl.kernel(
        _emb_kernel,
        out_type=jax.ShapeDtypeStruct((NW * N_CHUNKS, CHUNK, D_MODEL), jnp.float32),
        mesh=mesh,
        scratch_types=[
            pltpu.VMEM((N_CHUNKS, CHUNK), jnp.int32),
            pltpu.VMEM((CHUNK, D_MODEL), jnp.float32),
            pltpu.VMEM((CHUNK, D_MODEL), jnp.float32),
            pltpu.VMEM((CHUNK, D_MODEL), jnp.float32),
            pltpu.VMEM((CHUNK, D_MODEL), jnp.float32),
            pltpu.SemaphoreType.DMA,
            pltpu.SemaphoreType.DMA,
            pltpu.SemaphoreType.DMA,
            pltpu.SemaphoreType.DMA,
        ],
    )(x_g, emb_table)

    return out.reshape(s_dim, b_dim, D_MODEL).transpose(1, 0, 2)


# speedup vs baseline: 1.1444x; 1.1444x over previous
import math

import jax
import jax.numpy as jnp
from jax import lax
from jax.experimental import pallas as pl
from jax.experimental.pallas import tpu as pltpu
from jax.experimental.pallas import tpu_sc as plsc

D_MODEL = 128
SCALE = math.sqrt(D_MODEL)

NUM_CORES = 2
NUM_SUBCORES = 16
NW = NUM_CORES * NUM_SUBCORES

B_TOTAL = 4096 * 50
B_PER_W = B_TOTAL // NW
CHUNK = 128
N_CHUNKS = B_PER_W // CHUNK
NBUF = 2


def _emb_kernel(x_hbm, table_hbm, out_hbm,
                idx_v, gb0, gb1, ob0, ob1, gs0, gs1, os0, os1):
    gb = (gb0, gb1)
    ob = (ob0, ob1)
    gs = (gs0, gs1)
    osm = (os0, os1)

    wid = lax.axis_index("s") * NUM_CORES + lax.axis_index("c")

    pltpu.sync_copy(x_hbm.at[wid], idx_v)

    def g_desc(j, b):
        return pltpu.make_async_copy(table_hbm.at[idx_v.at[j]], gb[b], gs[b])

    def o_desc(j, b):
        return pltpu.make_async_copy(ob[b], out_hbm.at[wid * N_CHUNKS + j], osm[b])

    g_desc(0, 0).start()
    g_desc(1, 1).start()

    def step(j, b):
        g_desc(j, b).wait()

        @pl.when(j >= NBUF)
        def _():
            o_desc(j - NBUF, b).wait()

        @plsc.parallel_loop(0, CHUNK, unroll=4)
        def _(r):
            for c in range(D_MODEL // 16):
                ob[b][r, pl.ds(c * 16, 16)] = gb[b][r, pl.ds(c * 16, 16)] * SCALE

        o_desc(j, b).start()

        @pl.when(j + NBUF < N_CHUNKS)
        def _():
            g_desc(j + NBUF, b).start()

    @pl.loop(0, N_CHUNKS, step=NBUF)
    def _(j):
        for b in range(NBUF):
            step(j + b, b)

    o_desc(N_CHUNKS - NBUF, 0).wait()
    o_desc(N_CHUNKS - 1, 1).wait()


def kernel(x, emb_table):
    b_dim, s_dim = x.shape
    x_g = x.astype(jnp.int32).T.reshape(NW, N_CHUNKS, CHUNK)

    mesh = plsc.VectorSubcoreMesh(core_axis_name="c", subcore_axis_name="s")
    out = pl.kernel(
        _emb_kernel,
        out_type=jax.ShapeDtypeStruct((NW * N_CHUNKS, CHUNK, D_MODEL), jnp.float32),
        mesh=mesh,
        scratch_types=[
            pltpu.VMEM((N_CHUNKS, CHUNK), jnp.int32),
            pltpu.VMEM((CHUNK, D_MODEL), jnp.float32),
            pltpu.VMEM((CHUNK, D_MODEL), jnp.float32),
            pltpu.VMEM((CHUNK, D_MODEL), jnp.float32),
            pltpu.VMEM((CHUNK, D_MODEL), jnp.float32),
            pltpu.SemaphoreType.DMA,
            pltpu.SemaphoreType.DMA,
            pltpu.SemaphoreType.DMA,
            pltpu.SemaphoreType.DMA,
        ],
    )(x_g, emb_table)

    return out.reshape(s_dim, b_dim, D_MODEL).transpose(1, 0, 2)


# speedup vs baseline: 1.1535x; 1.0080x over previous
import math

import jax
import jax.numpy as jnp
from jax import lax
from jax.experimental import pallas as pl
from jax.experimental.pallas import tpu as pltpu
from jax.experimental.pallas import tpu_sc as plsc

D_MODEL = 128
SCALE = math.sqrt(D_MODEL)

NUM_CORES = 2
NUM_SUBCORES = 16
NW = NUM_CORES * NUM_SUBCORES

B_TOTAL = 4096 * 50
B_PER_W = B_TOTAL // NW
CHUNK = 128
N_CHUNKS = B_PER_W // CHUNK
NBUF = 2


def _emb_kernel(x_hbm, table_hbm, out_hbm,
                idx_v, gb0, gb1, ob0, ob1, gs0, gs1, os0, os1):
    gb = (gb0, gb1)
    ob = (ob0, ob1)
    gs = (gs0, gs1)
    osm = (os0, os1)

    wid = lax.axis_index("s") * NUM_CORES + lax.axis_index("c")

    pltpu.sync_copy(x_hbm.at[wid], idx_v)

    def g_desc(j, b):
        return pltpu.make_async_copy(table_hbm.at[idx_v.at[j]], gb[b], gs[b])

    def o_desc(j, b):
        return pltpu.make_async_copy(ob[b], out_hbm.at[wid * N_CHUNKS + j], osm[b])

    g_desc(0, 0).start()
    g_desc(1, 1).start()

    def step(j, b):
        g_desc(j, b).wait()

        @pl.when(j >= NBUF)
        def _():
            o_desc(j - NBUF, b).wait()

        @plsc.parallel_loop(0, CHUNK, unroll=4)
        def _(r):
            for c in range(D_MODEL // 16):
                ob[b][r, pl.ds(c * 16, 16)] = gb[b][r, pl.ds(c * 16, 16)] * SCALE

        @pl.when(j + NBUF < N_CHUNKS)
        def _():
            g_desc(j + NBUF, b).start()

        o_desc(j, b).start()

    @pl.loop(0, N_CHUNKS, step=NBUF)
    def _(j):
        for b in range(NBUF):
            step(j + b, b)

    o_desc(N_CHUNKS - NBUF, 0).wait()
    o_desc(N_CHUNKS - 1, 1).wait()


def kernel(x, emb_table):
    b_dim, s_dim = x.shape
    x_g = x.astype(jnp.int32).T.reshape(NW, N_CHUNKS, CHUNK)

    mesh = plsc.VectorSubcoreMesh(core_axis_name="c", subcore_axis_name="s")
    out = pl.kernel(
        _emb_kernel,
        out_type=jax.ShapeDtypeStruct((NW * N_CHUNKS, CHUNK, D_MODEL), jnp.float32),
        mesh=mesh,
        scratch_types=[
            pltpu.VMEM((N_CHUNKS, CHUNK), jnp.int32),
            pltpu.VMEM((CHUNK, D_MODEL), jnp.float32),
            pltpu.VMEM((CHUNK, D_MODEL), jnp.float32),
            pltpu.VMEM((CHUNK, D_MODEL), jnp.float32),
            pltpu.VMEM((CHUNK, D_MODEL), jnp.float32),
            pltpu.SemaphoreType.DMA,
            pltpu.SemaphoreType.DMA,
            pltpu.SemaphoreType.DMA,
            pltpu.SemaphoreType.DMA,
        ],
    )(x_g, emb_table)

    return out.reshape(s_dim, b_dim, D_MODEL).transpose(1, 0, 2)
